# baseline (speedup 1.0000x reference)
import functools

import jax
import jax.numpy as jnp
from jax.experimental import pallas as pl
from jax.experimental.pallas import tpu as pltpu

_NB = 4096
_NBUF = 6


def _w_copy(W_ref, wbuf, wsem, idx, slot, base, rows):
    return pltpu.make_async_copy(
        W_ref.at[pl.ds(base, rows), :],
        wbuf.at[idx, pl.ds(0, rows), :] if rows != _NB else wbuf.at[idx],
        wsem.at[slot],
    )


def _fc_kernel(x_ref, b_ref, W_ref, out_ref, wbuf, wsem, *, V, n, rem):
    x = x_ref[:]

    for k in range(_NBUF):
        _w_copy(W_ref, wbuf, wsem, k, k, k * _NB, _NB).start()

    def step(i, carry):
        m, s = carry
        slot = jax.lax.rem(i, _NBUF)
        _w_copy(W_ref, wbuf, wsem, slot, slot, i * _NB, _NB).wait()
        logits = jax.lax.dot_general(
            x, wbuf[slot],
            dimension_numbers=(((1,), (1,)), ((), ())),
            preferred_element_type=jnp.float32,
        ) + b_ref[:, pl.ds(i * _NB, _NB)]
        out_ref[:, pl.ds(i * _NB, _NB)] = logits

        m_blk = jnp.max(logits, axis=1, keepdims=True)
        m_new = jnp.maximum(m, m_blk)
        s_new = s * jnp.exp(m - m_new) + jnp.sum(
            jnp.exp(logits - m_new), axis=1, keepdims=True)

        nxt = i + _NBUF
        nslot = jax.lax.rem(nxt, _NBUF)

        @pl.when(nxt < n - 1)
        def _():
            _w_copy(W_ref, wbuf, wsem, nslot, nslot, nxt * _NB, _NB).start()

        @pl.when(nxt == n - 1)
        def _():
            _w_copy(W_ref, wbuf, wsem, nslot, nslot, nxt * _NB, rem).start()

        return m_new, s_new

    m0 = jnp.full((x.shape[0], 1), -jnp.inf, dtype=jnp.float32)
    s0 = jnp.zeros((x.shape[0], 1), dtype=jnp.float32)
    m, s = jax.lax.fori_loop(0, n - 1, step, (m0, s0))

    lslot = (n - 1) % _NBUF
    _w_copy(W_ref, wbuf, wsem, lslot, lslot, (n - 1) * _NB, rem).wait()
    logits = jax.lax.dot_general(
        x, wbuf[lslot, :rem, :],
        dimension_numbers=(((1,), (1,)), ((), ())),
        preferred_element_type=jnp.float32,
    ) + b_ref[:, pl.ds((n - 1) * _NB, rem)]
    out_ref[:, pl.ds((n - 1) * _NB, rem)] = logits
    m_blk = jnp.max(logits, axis=1, keepdims=True)
    m_new = jnp.maximum(m, m_blk)
    s = s * jnp.exp(m - m_new) + jnp.sum(
        jnp.exp(logits - m_new), axis=1, keepdims=True)
    lse = m_new + jnp.log(s)

    out_ref[:, :] = out_ref[:, :] - lse


@jax.jit
def kernel(x, W, b):
    B, K = x.shape
    V = W.shape[0]
    n = pl.cdiv(V, _NB)
    rem = V - (n - 1) * _NB
    b2 = b.reshape(1, V)

    return pl.pallas_call(
        functools.partial(_fc_kernel, V=V, n=n, rem=rem),
        in_specs=[
            pl.BlockSpec(memory_space=pltpu.VMEM),
            pl.BlockSpec(memory_space=pltpu.VMEM),
            pl.BlockSpec(memory_space=pl.ANY),
        ],
        out_specs=pl.BlockSpec(memory_space=pltpu.VMEM),
        out_shape=jax.ShapeDtypeStruct((B, V), jnp.float32),
        scratch_shapes=[
            pltpu.VMEM((_NBUF, _NB, K), jnp.float32),
            pltpu.SemaphoreType.DMA((_NBUF,)),
        ],
    )(x, b2, W)


# speedup vs baseline: 1.0167x; 1.0167x over previous
import functools

import jax
import jax.numpy as jnp
from jax.experimental import pallas as pl
from jax.experimental.pallas import tpu as pltpu

_NB = 4096
_NBUF = 5


def _w_copy(W_ref, wbuf, wsem, idx, slot, base, rows):
    return pltpu.make_async_copy(
        W_ref.at[pl.ds(base, rows), :],
        wbuf.at[idx, pl.ds(0, rows), :] if rows != _NB else wbuf.at[idx],
        wsem.at[slot],
    )


def _fc_kernel(x_ref, b_ref, W_ref, out_ref, wbuf, wsem, *, V, n, rem):
    x = x_ref[:]

    for k in range(_NBUF):
        _w_copy(W_ref, wbuf, wsem, k, k, k * _NB, _NB).start()

    def step(i, carry):
        m, s = carry
        slot = jax.lax.rem(i, _NBUF)
        _w_copy(W_ref, wbuf, wsem, slot, slot, i * _NB, _NB).wait()
        logits = jax.lax.dot_general(
            x, wbuf[slot],
            dimension_numbers=(((1,), (1,)), ((), ())),
            preferred_element_type=jnp.float32,
        ) + b_ref[:, pl.ds(i * _NB, _NB)]
        out_ref[:, pl.ds(i * _NB, _NB)] = logits

        m_blk = jnp.max(logits, axis=1, keepdims=True)
        m_new = jnp.maximum(m, m_blk)
        s_new = s * jnp.exp(m - m_new) + jnp.sum(
            jnp.exp(logits - m_new), axis=1, keepdims=True)

        nxt = i + _NBUF
        nslot = jax.lax.rem(nxt, _NBUF)

        @pl.when(nxt < n - 1)
        def _():
            _w_copy(W_ref, wbuf, wsem, nslot, nslot, nxt * _NB, _NB).start()

        @pl.when(nxt == n - 1)
        def _():
            _w_copy(W_ref, wbuf, wsem, nslot, nslot, nxt * _NB, rem).start()

        return m_new, s_new

    m0 = jnp.full((x.shape[0], 1), -jnp.inf, dtype=jnp.float32)
    s0 = jnp.zeros((x.shape[0], 1), dtype=jnp.float32)
    m, s = jax.lax.fori_loop(0, n - 1, step, (m0, s0))

    lslot = (n - 1) % _NBUF
    _w_copy(W_ref, wbuf, wsem, lslot, lslot, (n - 1) * _NB, rem).wait()
    logits = jax.lax.dot_general(
        x, wbuf[lslot, :rem, :],
        dimension_numbers=(((1,), (1,)), ((), ())),
        preferred_element_type=jnp.float32,
    ) + b_ref[:, pl.ds((n - 1) * _NB, rem)]
    out_ref[:, pl.ds((n - 1) * _NB, rem)] = logits
    m_blk = jnp.max(logits, axis=1, keepdims=True)
    m_new = jnp.maximum(m, m_blk)
    s = s * jnp.exp(m - m_new) + jnp.sum(
        jnp.exp(logits - m_new), axis=1, keepdims=True)
    lse = m_new + jnp.log(s)

    out_ref[:, :] = out_ref[:, :] - lse


@jax.jit
def kernel(x, W, b):
    B, K = x.shape
    V = W.shape[0]
    n = pl.cdiv(V, _NB)
    rem = V - (n - 1) * _NB
    b2 = b.reshape(1, V)

    return pl.pallas_call(
        functools.partial(_fc_kernel, V=V, n=n, rem=rem),
        in_specs=[
            pl.BlockSpec(memory_space=pltpu.VMEM),
            pl.BlockSpec(memory_space=pltpu.VMEM),
            pl.BlockSpec(memory_space=pl.ANY),
        ],
        out_specs=pl.BlockSpec(memory_space=pltpu.VMEM),
        out_shape=jax.ShapeDtypeStruct((B, V), jnp.float32),
        scratch_shapes=[
            pltpu.VMEM((_NBUF, _NB, K), jnp.float32),
            pltpu.SemaphoreType.DMA((_NBUF,)),
        ],
    )(x, b2, W)


# speedup vs baseline: 1.0553x; 1.0380x over previous
import functools

import jax
import jax.numpy as jnp
from jax.experimental import pallas as pl
from jax.experimental.pallas import tpu as pltpu

_NB = 4096
_NBUF = 4


def _w_copy(W_ref, wbuf, wsem, idx, slot, base, rows):
    return pltpu.make_async_copy(
        W_ref.at[pl.ds(base, rows), :],
        wbuf.at[idx, pl.ds(0, rows), :] if rows != _NB else wbuf.at[idx],
        wsem.at[slot],
    )


def _fc_kernel(x_ref, b_ref, W_ref, out_ref, wbuf, wsem, *, V, n, rem):
    x = x_ref[:]

    for k in range(_NBUF):
        _w_copy(W_ref, wbuf, wsem, k, k, k * _NB, _NB).start()

    def step(i, carry):
        m, s = carry
        slot = jax.lax.rem(i, _NBUF)
        _w_copy(W_ref, wbuf, wsem, slot, slot, i * _NB, _NB).wait()
        logits = jax.lax.dot_general(
            x, wbuf[slot],
            dimension_numbers=(((1,), (1,)), ((), ())),
            preferred_element_type=jnp.float32,
        ) + b_ref[:, pl.ds(i * _NB, _NB)]
        out_ref[:, pl.ds(i * _NB, _NB)] = logits

        m_blk = jnp.max(logits, axis=1, keepdims=True)
        m_new = jnp.maximum(m, m_blk)
        s_new = s * jnp.exp(m - m_new) + jnp.sum(
            jnp.exp(logits - m_new), axis=1, keepdims=True)

        nxt = i + _NBUF
        nslot = jax.lax.rem(nxt, _NBUF)

        @pl.when(nxt < n - 1)
        def _():
            _w_copy(W_ref, wbuf, wsem, nslot, nslot, nxt * _NB, _NB).start()

        @pl.when(nxt == n - 1)
        def _():
            _w_copy(W_ref, wbuf, wsem, nslot, nslot, nxt * _NB, rem).start()

        return m_new, s_new

    m0 = jnp.full((x.shape[0], 1), -jnp.inf, dtype=jnp.float32)
    s0 = jnp.zeros((x.shape[0], 1), dtype=jnp.float32)
    m, s = jax.lax.fori_loop(0, n - 1, step, (m0, s0))

    lslot = (n - 1) % _NBUF
    _w_copy(W_ref, wbuf, wsem, lslot, lslot, (n - 1) * _NB, rem).wait()
    logits = jax.lax.dot_general(
        x, wbuf[lslot, :rem, :],
        dimension_numbers=(((1,), (1,)), ((), ())),
        preferred_element_type=jnp.float32,
    ) + b_ref[:, pl.ds((n - 1) * _NB, rem)]
    out_ref[:, pl.ds((n - 1) * _NB, rem)] = logits
    m_blk = jnp.max(logits, axis=1, keepdims=True)
    m_new = jnp.maximum(m, m_blk)
    s = s * jnp.exp(m - m_new) + jnp.sum(
        jnp.exp(logits - m_new), axis=1, keepdims=True)
    lse = m_new + jnp.log(s)

    out_ref[:, :] = out_ref[:, :] - lse


@jax.jit
def kernel(x, W, b):
    B, K = x.shape
    V = W.shape[0]
    n = pl.cdiv(V, _NB)
    rem = V - (n - 1) * _NB
    b2 = b.reshape(1, V)

    return pl.pallas_call(
        functools.partial(_fc_kernel, V=V, n=n, rem=rem),
        in_specs=[
            pl.BlockSpec(memory_space=pltpu.VMEM),
            pl.BlockSpec(memory_space=pltpu.VMEM),
            pl.BlockSpec(memory_space=pl.ANY),
        ],
        out_specs=pl.BlockSpec(memory_space=pltpu.VMEM),
        out_shape=jax.ShapeDtypeStruct((B, V), jnp.float32),
        scratch_shapes=[
            pltpu.VMEM((_NBUF, _NB, K), jnp.float32),
            pltpu.SemaphoreType.DMA((_NBUF,)),
        ],
    )(x, b2, W)
